# Initial kernel scaffold; baseline (speedup 1.0000x reference)
import functools

import jax
import jax.numpy as jnp
from jax import lax
from jax.experimental import pallas as pl
from jax.experimental.pallas import tpu as pltpu
from jax.experimental.pallas import tpu_sc as plsc

N_NODES = 10000
N_EDGES = 160000
D_IN = 32
D_OUT = 16
PACK = 32

NC = 2
NS = 16
NW = NC * NS
CHUNK = 128
BE = 2048
E_PAD = 163840
EPW = E_PAD // NW
NCHUNKS = EPW // CHUNK
ROWS_PER_TILE = N_NODES // NS

_mesh = plsc.VectorSubcoreMesh(core_axis_name="c", subcore_axis_name="s")


def _node_proj_body(nf_ref, wu_ref, u_ref):
    u_ref[...] = jnp.dot(nf_ref[...], wu_ref[...],
                         preferred_element_type=jnp.float32)


def _edge_body(xs_ref, qr_ref, es_ref, sh_ref, wk1_ref, wk2_ref,
               wv1_ref, wv2_ref, r_ref, s_ref, out_ref):
    i = pl.program_id(0)
    xs = xs_ref[...] * sh_ref[...]
    xrep = jnp.dot(xs, r_ref[...], preferred_element_type=jnp.float32)
    es = es_ref[...]
    hk = jax.nn.silu(jnp.dot(es, wk1_ref[...],
                             preferred_element_type=jnp.float32) * 0.25)
    wk = jnp.dot(hk, wk2_ref[...], preferred_element_type=jnp.float32)
    kk = jnp.dot(xrep * wk, s_ref[...],
                 preferred_element_type=jnp.float32) * (1.0 / 32.0)
    hv = jax.nn.silu(jnp.dot(es, wv1_ref[...],
                             preferred_element_type=jnp.float32) * 0.25)
    wv = jnp.dot(hv, wv2_ref[...], preferred_element_type=jnp.float32)
    vv = jnp.dot(xrep * wv, s_ref[...],
                 preferred_element_type=jnp.float32) * (1.0 / 32.0)
    logit = jnp.sum(qr_ref[...] * kk, axis=1, keepdims=True)
    rows = jax.lax.broadcasted_iota(jnp.int32, (BE, 1), 0) + i * BE
    valid = rows < N_EDGES
    expv = jnp.where(valid, jnp.exp(logit), 0.0)
    sv = jnp.where(valid, jnp.exp(0.5 * logit), 0.0) * vv
    out_ref[...] = jnp.concatenate(
        [expv, sv, jnp.zeros((BE, PACK - 1 - D_OUT), jnp.float32)], axis=1)


def _final_body(p0_ref, p1_ref, out_ref):
    t = p0_ref[...] + p1_ref[...]
    z = t[:, 0:1]
    s = t[:, 1:1 + D_OUT]
    out_ref[...] = jnp.where(z > 0.0,
                             s * jax.lax.rsqrt(jnp.maximum(z, 1e-30)), 0.0)


@functools.partial(
    pl.kernel,
    out_type=[jax.ShapeDtypeStruct((E_PAD, D_IN), jnp.float32),
              jax.ShapeDtypeStruct((E_PAD, D_OUT), jnp.float32)],
    mesh=_mesh,
    scratch_types=[pltpu.VMEM((NCHUNKS, CHUNK), jnp.int32),
                   pltpu.VMEM((NCHUNKS, CHUNK), jnp.int32),
                   pltpu.VMEM((CHUNK, D_IN), jnp.float32),
                   pltpu.VMEM((CHUNK, D_OUT), jnp.float32),
                   pltpu.SemaphoreType.DMA,
                   pltpu.SemaphoreType.DMA],
)
def _sc_gather(node_hbm, u_hbm, snd_hbm, rcv_hbm, xs_out, qr_out,
               sidx, ridx, xbuf, qbuf, sem1, sem2):
    wid = lax.axis_index("s") * NC + lax.axis_index("c")
    pltpu.sync_copy(snd_hbm.at[pl.ds(wid * NCHUNKS, NCHUNKS)], sidx)
    pltpu.sync_copy(rcv_hbm.at[pl.ds(wid * NCHUNKS, NCHUNKS)], ridx)

    def body(j, carry):
        base = wid * EPW + j * CHUNK
        pltpu.async_copy(node_hbm.at[sidx.at[j]], xbuf, sem1).wait()
        pltpu.sync_copy(xbuf, xs_out.at[pl.ds(base, CHUNK)])
        pltpu.async_copy(u_hbm.at[ridx.at[j]], qbuf, sem2).wait()
        pltpu.sync_copy(qbuf, qr_out.at[pl.ds(base, CHUNK)])
        return carry

    lax.fori_loop(0, NCHUNKS, body, 0)


@functools.partial(
    pl.kernel,
    out_type=jax.ShapeDtypeStruct((NC, N_NODES, PACK), jnp.float32),
    mesh=_mesh,
    scratch_types=[pltpu.VMEM((NCHUNKS, CHUNK), jnp.int32),
                   pltpu.VMEM((CHUNK, PACK), jnp.float32),
                   pltpu.VMEM((ROWS_PER_TILE, PACK), jnp.float32),
                   pltpu.VMEM_SHARED((N_NODES, PACK), jnp.float32)],
)
def _sc_scatter(packed_hbm, rcv_hbm, zeros_hbm, part_out,
                ridx, vbuf, dbuf, accum):
    cid = lax.axis_index("c")
    sid = lax.axis_index("s")
    wid = sid * NC + cid
    pltpu.sync_copy(zeros_hbm, dbuf)
    pltpu.sync_copy(dbuf, accum.at[pl.ds(sid * ROWS_PER_TILE, ROWS_PER_TILE)])
    plsc.subcore_barrier()
    pltpu.sync_copy(rcv_hbm.at[pl.ds(wid * NCHUNKS, NCHUNKS)], ridx)

    def body(j, carry):
        pltpu.sync_copy(packed_hbm.at[pl.ds(wid * EPW + j * CHUNK, CHUNK)],
                        vbuf)
        pltpu.sync_copy(vbuf, accum.at[ridx.at[j]], add=True)
        return carry

    lax.fori_loop(0, NCHUNKS, body, 0)
    plsc.subcore_barrier()
    pltpu.sync_copy(accum.at[pl.ds(sid * ROWS_PER_TILE, ROWS_PER_TILE)], dbuf)
    pltpu.sync_copy(dbuf, part_out.at[cid, pl.ds(sid * ROWS_PER_TILE,
                                                 ROWS_PER_TILE)])


def kernel(node_ft, edge_index, edge_sh, edge_scalars,
           Wq, Wk1, Wk2, Wv1, Wv2, Wdot):
    pad = E_PAD - N_EDGES
    snd = jnp.pad(edge_index[0], (0, pad)).astype(jnp.int32)
    rcv = jnp.pad(edge_index[1], (0, pad)).astype(jnp.int32)
    snd2d = snd.reshape(E_PAD // CHUNK, CHUNK)
    rcv2d = rcv.reshape(E_PAD // CHUNK, CHUNK)
    es_p = jnp.pad(edge_scalars, ((0, pad), (0, 0)))
    sh_p = jnp.pad(edge_sh, ((0, pad), (0, 0)))

    Wu = (Wq @ Wdot) * (1.0 / (jnp.sqrt(32.0) * 16.0))
    Rm = jnp.repeat(jnp.eye(D_IN, dtype=jnp.float32), D_OUT, axis=1)
    Sm = jnp.tile(jnp.eye(D_OUT, dtype=jnp.float32), (D_IN, 1))
    zeros = jnp.zeros((ROWS_PER_TILE, PACK), jnp.float32)

    u = pl.pallas_call(
        _node_proj_body,
        out_shape=jax.ShapeDtypeStruct((N_NODES, 16), jnp.float32),
    )(node_ft, Wu)

    xs_g, qr_g = _sc_gather(node_ft, u, snd2d, rcv2d)

    grid = (E_PAD // BE,)
    packed = pl.pallas_call(
        _edge_body,
        grid=grid,
        in_specs=[
            pl.BlockSpec((BE, D_IN), lambda i: (i, 0)),
            pl.BlockSpec((BE, D_OUT), lambda i: (i, 0)),
            pl.BlockSpec((BE, 16), lambda i: (i, 0)),
            pl.BlockSpec((BE, 1), lambda i: (i, 0)),
            pl.BlockSpec((16, 32), lambda i: (0, 0)),
            pl.BlockSpec((32, 512), lambda i: (0, 0)),
            pl.BlockSpec((16, 32), lambda i: (0, 0)),
            pl.BlockSpec((32, 512), lambda i: (0, 0)),
            pl.BlockSpec((32, 512), lambda i: (0, 0)),
            pl.BlockSpec((512, 16), lambda i: (0, 0)),
        ],
        out_specs=pl.BlockSpec((BE, PACK), lambda i: (i, 0)),
        out_shape=jax.ShapeDtypeStruct((E_PAD, PACK), jnp.float32),
    )(xs_g, qr_g, es_p, sh_p, Wk1, Wk2, Wv1, Wv2, Rm, Sm)

    part = _sc_scatter(packed, rcv2d, zeros)

    out = pl.pallas_call(
        _final_body,
        out_shape=jax.ShapeDtypeStruct((N_NODES, D_OUT), jnp.float32),
    )(part[0], part[1])
    return out


# speedup vs baseline: 3.3145x; 3.3145x over previous
import functools

import jax
import jax.numpy as jnp
from jax import lax
from jax.experimental import pallas as pl
from jax.experimental.pallas import tpu as pltpu
from jax.experimental.pallas import tpu_sc as plsc

N_NODES = 10000
N_EDGES = 160000
D_IN = 32
D_OUT = 16
PACK = 32

NC = 2
NS = 16
NW = NC * NS
CHUNK = 128
BE = 2048
E_PAD = 163840
EPW = E_PAD // NW
NCHUNKS = EPW // CHUNK
ROWS_PER_TILE = N_NODES // NS

_mesh = plsc.VectorSubcoreMesh(core_axis_name="c", subcore_axis_name="s")


def _node_proj_body(nf_ref, wu_ref, u_ref):
    u_ref[...] = jnp.dot(nf_ref[...], wu_ref[...],
                         preferred_element_type=jnp.float32)


def _edge_body(xs_ref, qr_ref, es_ref, sh_ref, wk1_ref, wk2_ref,
               wv1_ref, wv2_ref, r_ref, s_ref, out_ref):
    i = pl.program_id(0)
    xs = xs_ref[...] * sh_ref[...]
    xrep = jnp.dot(xs, r_ref[...], preferred_element_type=jnp.float32)
    es = es_ref[...]
    hk = jax.nn.silu(jnp.dot(es, wk1_ref[...],
                             preferred_element_type=jnp.float32) * 0.25)
    wk = jnp.dot(hk, wk2_ref[...], preferred_element_type=jnp.float32)
    kk = jnp.dot(xrep * wk, s_ref[...],
                 preferred_element_type=jnp.float32) * (1.0 / 32.0)
    hv = jax.nn.silu(jnp.dot(es, wv1_ref[...],
                             preferred_element_type=jnp.float32) * 0.25)
    wv = jnp.dot(hv, wv2_ref[...], preferred_element_type=jnp.float32)
    vv = jnp.dot(xrep * wv, s_ref[...],
                 preferred_element_type=jnp.float32) * (1.0 / 32.0)
    logit = jnp.sum(qr_ref[...] * kk, axis=1, keepdims=True)
    rows = jax.lax.broadcasted_iota(jnp.int32, (BE, 1), 0) + i * BE
    valid = rows < N_EDGES
    expv = jnp.where(valid, jnp.exp(logit), 0.0)
    sv = jnp.where(valid, jnp.exp(0.5 * logit), 0.0) * vv
    out_ref[...] = jnp.concatenate(
        [expv, sv, jnp.zeros((BE, PACK - 1 - D_OUT), jnp.float32)], axis=1)


def _final_body(p0_ref, p1_ref, out_ref):
    t = p0_ref[...] + p1_ref[...]
    z = t[:, 0:1]
    s = t[:, 1:1 + D_OUT]
    out_ref[...] = jnp.where(z > 0.0,
                             s * jax.lax.rsqrt(jnp.maximum(z, 1e-30)), 0.0)


@functools.partial(
    pl.kernel,
    out_type=[jax.ShapeDtypeStruct((E_PAD, D_IN), jnp.float32),
              jax.ShapeDtypeStruct((E_PAD, D_OUT), jnp.float32)],
    mesh=_mesh,
    scratch_types=[pltpu.VMEM((NCHUNKS, CHUNK), jnp.int32),
                   pltpu.VMEM((NCHUNKS, CHUNK), jnp.int32),
                   pltpu.VMEM((CHUNK, D_IN), jnp.float32),
                   pltpu.VMEM((CHUNK, D_OUT), jnp.float32),
                   pltpu.SemaphoreType.DMA,
                   pltpu.SemaphoreType.DMA],
    compiler_params=pltpu.CompilerParams(use_tc_tiling_on_sc=False),
)
def _sc_gather(node_hbm, u_hbm, snd_hbm, rcv_hbm, xs_out, qr_out,
               sidx, ridx, xbuf, qbuf, sem1, sem2):
    wid = lax.axis_index("s") * NC + lax.axis_index("c")
    pltpu.sync_copy(snd_hbm.at[pl.ds(wid * NCHUNKS, NCHUNKS)], sidx)
    pltpu.sync_copy(rcv_hbm.at[pl.ds(wid * NCHUNKS, NCHUNKS)], ridx)

    def body(j, carry):
        base = wid * EPW + j * CHUNK
        pltpu.async_copy(node_hbm.at[sidx.at[j]], xbuf, sem1).wait()
        pltpu.sync_copy(xbuf, xs_out.at[pl.ds(base, CHUNK)])
        pltpu.async_copy(u_hbm.at[ridx.at[j]], qbuf, sem2).wait()
        pltpu.sync_copy(qbuf, qr_out.at[pl.ds(base, CHUNK)])
        return carry

    lax.fori_loop(0, NCHUNKS, body, 0)


@functools.partial(
    pl.kernel,
    out_type=jax.ShapeDtypeStruct((NC, N_NODES, PACK), jnp.float32),
    mesh=_mesh,
    scratch_types=[pltpu.VMEM((NCHUNKS, CHUNK), jnp.int32),
                   pltpu.VMEM((CHUNK, PACK), jnp.float32),
                   pltpu.VMEM((ROWS_PER_TILE, PACK), jnp.float32),
                   pltpu.VMEM_SHARED((N_NODES, PACK), jnp.float32)],
    compiler_params=pltpu.CompilerParams(use_tc_tiling_on_sc=False),
)
def _sc_scatter(packed_hbm, rcv_hbm, zeros_hbm, part_out,
                ridx, vbuf, dbuf, accum):
    cid = lax.axis_index("c")
    sid = lax.axis_index("s")
    wid = sid * NC + cid
    pltpu.sync_copy(zeros_hbm, dbuf)
    pltpu.sync_copy(dbuf, accum.at[pl.ds(sid * ROWS_PER_TILE, ROWS_PER_TILE)])
    plsc.subcore_barrier()
    pltpu.sync_copy(rcv_hbm.at[pl.ds(wid * NCHUNKS, NCHUNKS)], ridx)

    def body(j, carry):
        pltpu.sync_copy(packed_hbm.at[pl.ds(wid * EPW + j * CHUNK, CHUNK)],
                        vbuf)
        pltpu.sync_copy(vbuf, accum.at[ridx.at[j]], add=True)
        return carry

    lax.fori_loop(0, NCHUNKS, body, 0)
    plsc.subcore_barrier()
    pltpu.sync_copy(accum.at[pl.ds(sid * ROWS_PER_TILE, ROWS_PER_TILE)], dbuf)
    pltpu.sync_copy(dbuf, part_out.at[cid, pl.ds(sid * ROWS_PER_TILE,
                                                 ROWS_PER_TILE)])


def kernel(node_ft, edge_index, edge_sh, edge_scalars,
           Wq, Wk1, Wk2, Wv1, Wv2, Wdot):
    pad = E_PAD - N_EDGES
    snd = jnp.pad(edge_index[0], (0, pad)).astype(jnp.int32)
    rcv = jnp.pad(edge_index[1], (0, pad)).astype(jnp.int32)
    snd2d = snd.reshape(E_PAD // CHUNK, CHUNK)
    rcv2d = rcv.reshape(E_PAD // CHUNK, CHUNK)
    es_p = jnp.pad(edge_scalars, ((0, pad), (0, 0)))
    sh_p = jnp.pad(edge_sh, ((0, pad), (0, 0)))

    Wu = (Wq @ Wdot) * (1.0 / (jnp.sqrt(32.0) * 16.0))
    Rm = jnp.repeat(jnp.eye(D_IN, dtype=jnp.float32), D_OUT, axis=1)
    Sm = jnp.tile(jnp.eye(D_OUT, dtype=jnp.float32), (D_IN, 1))
    zeros = jnp.zeros((ROWS_PER_TILE, PACK), jnp.float32)

    u = pl.pallas_call(
        _node_proj_body,
        out_shape=jax.ShapeDtypeStruct((N_NODES, 16), jnp.float32),
    )(node_ft, Wu)

    xs_g, qr_g = _sc_gather(node_ft, u, snd2d, rcv2d)

    grid = (E_PAD // BE,)
    packed = pl.pallas_call(
        _edge_body,
        grid=grid,
        in_specs=[
            pl.BlockSpec((BE, D_IN), lambda i: (i, 0)),
            pl.BlockSpec((BE, D_OUT), lambda i: (i, 0)),
            pl.BlockSpec((BE, 16), lambda i: (i, 0)),
            pl.BlockSpec((BE, 1), lambda i: (i, 0)),
            pl.BlockSpec((16, 32), lambda i: (0, 0)),
            pl.BlockSpec((32, 512), lambda i: (0, 0)),
            pl.BlockSpec((16, 32), lambda i: (0, 0)),
            pl.BlockSpec((32, 512), lambda i: (0, 0)),
            pl.BlockSpec((32, 512), lambda i: (0, 0)),
            pl.BlockSpec((512, 16), lambda i: (0, 0)),
        ],
        out_specs=pl.BlockSpec((BE, PACK), lambda i: (i, 0)),
        out_shape=jax.ShapeDtypeStruct((E_PAD, PACK), jnp.float32),
    )(xs_g, qr_g, es_p, sh_p, Wk1, Wk2, Wv1, Wv2, Rm, Sm)

    part = _sc_scatter(packed, rcv2d, zeros)

    out = pl.pallas_call(
        _final_body,
        out_shape=jax.ShapeDtypeStruct((N_NODES, D_OUT), jnp.float32),
    )(part[0], part[1])
    return out


# speedup vs baseline: 4.1563x; 1.2539x over previous
import functools

import jax
import jax.numpy as jnp
from jax import lax
from jax.experimental import pallas as pl
from jax.experimental.pallas import tpu as pltpu
from jax.experimental.pallas import tpu_sc as plsc

N_NODES = 10000
N_EDGES = 160000
D_IN = 32
D_OUT = 16
PACK = 32

NC = 2
NS = 16
NW = NC * NS
CHUNK = 125
BE = 2000
EPW = N_EDGES // NW
NCHUNKS = EPW // CHUNK
KG = 4
NGROUPS = NCHUNKS // KG
ROWS_PER_TILE = N_NODES // NS

_mesh = plsc.VectorSubcoreMesh(core_axis_name="c", subcore_axis_name="s")


def _node_proj_body(nf_ref, wu_ref, u_ref):
    u_ref[...] = jnp.dot(nf_ref[...], wu_ref[...],
                         preferred_element_type=jnp.float32)


def _edge_body(xs_ref, qr_ref, es_ref, sh_ref, wk1_ref, wk2_ref,
               wv1_ref, wv2_ref, r_ref, s_ref, out_ref):
    xs = xs_ref[...] * sh_ref[...]
    xrep = jnp.dot(xs, r_ref[...], preferred_element_type=jnp.float32)
    es = es_ref[...]
    hk = jax.nn.silu(jnp.dot(es, wk1_ref[...],
                             preferred_element_type=jnp.float32) * 0.25)
    wk = jnp.dot(hk, wk2_ref[...], preferred_element_type=jnp.float32)
    kk = jnp.dot(xrep * wk, s_ref[...],
                 preferred_element_type=jnp.float32) * (1.0 / 32.0)
    hv = jax.nn.silu(jnp.dot(es, wv1_ref[...],
                             preferred_element_type=jnp.float32) * 0.25)
    wv = jnp.dot(hv, wv2_ref[...], preferred_element_type=jnp.float32)
    vv = jnp.dot(xrep * wv, s_ref[...],
                 preferred_element_type=jnp.float32) * (1.0 / 32.0)
    logit = jnp.sum(qr_ref[...] * kk, axis=1, keepdims=True)
    expv = jnp.exp(logit)
    sv = jnp.exp(0.5 * logit) * vv
    out_ref[...] = jnp.concatenate(
        [expv, sv, jnp.zeros((BE, PACK - 1 - D_OUT), jnp.float32)], axis=1)


def _final_body(p0_ref, p1_ref, out_ref):
    t = p0_ref[...] + p1_ref[...]
    z = t[:, 0:1]
    s = t[:, 1:1 + D_OUT]
    out_ref[...] = jnp.where(z > 0.0,
                             s * jax.lax.rsqrt(jnp.maximum(z, 1e-30)), 0.0)


@functools.partial(
    pl.kernel,
    out_type=[jax.ShapeDtypeStruct((N_EDGES, D_IN), jnp.float32),
              jax.ShapeDtypeStruct((N_EDGES, D_OUT), jnp.float32)],
    mesh=_mesh,
    scratch_types=[pltpu.VMEM((NCHUNKS, CHUNK), jnp.int32),
                   pltpu.VMEM((NCHUNKS, CHUNK), jnp.int32),
                   pltpu.VMEM((2, KG, CHUNK, D_IN), jnp.float32),
                   pltpu.VMEM((2, KG, CHUNK, D_OUT), jnp.float32),
                   pltpu.SemaphoreType.DMA,
                   pltpu.SemaphoreType.DMA,
                   pltpu.SemaphoreType.DMA,
                   pltpu.SemaphoreType.DMA],
    compiler_params=pltpu.CompilerParams(use_tc_tiling_on_sc=False),
)
def _sc_gather(node_hbm, u_hbm, snd_hbm, rcv_hbm, xs_out, qr_out,
               sidx, ridx, xbuf, qbuf, gsemA, gsemB, ssemA, ssemB):
    wid = lax.axis_index("s") * NC + lax.axis_index("c")
    pltpu.sync_copy(snd_hbm.at[pl.ds(wid * NCHUNKS, NCHUNKS)], sidx)
    pltpu.sync_copy(rcv_hbm.at[pl.ds(wid * NCHUNKS, NCHUNKS)], ridx)
    ebase = wid * EPW

    def fire_gathers(g, bank, gsem):
        for b in range(KG):
            j = g * KG + b
            pltpu.async_copy(node_hbm.at[sidx.at[j]], xbuf.at[bank, b], gsem)
            pltpu.async_copy(u_hbm.at[ridx.at[j]], qbuf.at[bank, b], gsem)

    def drain_gathers(g, bank, gsem):
        for b in range(KG):
            j = g * KG + b
            pltpu.make_async_copy(node_hbm.at[sidx.at[j]], xbuf.at[bank, b],
                                  gsem).wait()
            pltpu.make_async_copy(u_hbm.at[ridx.at[j]], qbuf.at[bank, b],
                                  gsem).wait()

    def fire_stores(g, bank, ssem):
        for b in range(KG):
            j = g * KG + b
            dst = xs_out.at[pl.ds(ebase + j * CHUNK, CHUNK)]
            pltpu.async_copy(xbuf.at[bank, b], dst, ssem)
            dq = qr_out.at[pl.ds(ebase + j * CHUNK, CHUNK)]
            pltpu.async_copy(qbuf.at[bank, b], dq, ssem)

    def drain_stores(g, bank, ssem):
        for b in range(KG):
            j = g * KG + b
            dst = xs_out.at[pl.ds(ebase + j * CHUNK, CHUNK)]
            pltpu.make_async_copy(xbuf.at[bank, b], dst, ssem).wait()
            dq = qr_out.at[pl.ds(ebase + j * CHUNK, CHUNK)]
            pltpu.make_async_copy(qbuf.at[bank, b], dq, ssem).wait()

    fire_gathers(0, 0, gsemA)

    def body(t, carry):
        gA = 2 * t
        gB = 2 * t + 1
        drain_gathers(gA, 0, gsemA)
        fire_stores(gA, 0, ssemA)

        @pl.when(t > 0)
        def _():
            drain_stores(gB - 2, 1, ssemB)

        fire_gathers(gB, 1, gsemB)
        drain_stores(gA, 0, ssemA)

        @pl.when(t + 1 < NGROUPS // 2)
        def _():
            fire_gathers(gA + 2, 0, gsemA)

        drain_gathers(gB, 1, gsemB)
        fire_stores(gB, 1, ssemB)
        return carry

    lax.fori_loop(0, NGROUPS // 2, body, 0)
    drain_stores(NGROUPS - 1, 1, ssemB)


@functools.partial(
    pl.kernel,
    out_type=jax.ShapeDtypeStruct((NC, N_NODES, PACK), jnp.float32),
    mesh=_mesh,
    scratch_types=[pltpu.VMEM((NCHUNKS, CHUNK), jnp.int32),
                   pltpu.VMEM((2, CHUNK, PACK), jnp.float32),
                   pltpu.VMEM((ROWS_PER_TILE, PACK), jnp.float32),
                   pltpu.VMEM_SHARED((N_NODES, PACK), jnp.float32),
                   pltpu.SemaphoreType.DMA,
                   pltpu.SemaphoreType.DMA],
    compiler_params=pltpu.CompilerParams(use_tc_tiling_on_sc=False),
)
def _sc_scatter(packed_hbm, rcv_hbm, zeros_hbm, part_out,
                ridx, vbuf, dbuf, accum, lsemA, lsemB):
    cid = lax.axis_index("c")
    sid = lax.axis_index("s")
    wid = sid * NC + cid
    pltpu.sync_copy(zeros_hbm, dbuf)
    pltpu.sync_copy(dbuf, accum.at[pl.ds(sid * ROWS_PER_TILE, ROWS_PER_TILE)])
    plsc.subcore_barrier()
    pltpu.sync_copy(rcv_hbm.at[pl.ds(wid * NCHUNKS, NCHUNKS)], ridx)
    ebase = wid * EPW

    def fire_load(j, bank, sem):
        pltpu.async_copy(packed_hbm.at[pl.ds(ebase + j * CHUNK, CHUNK)],
                         vbuf.at[bank], sem)

    def drain_load(j, bank, sem):
        pltpu.make_async_copy(packed_hbm.at[pl.ds(ebase + j * CHUNK, CHUNK)],
                              vbuf.at[bank], sem).wait()

    fire_load(0, 0, lsemA)

    def body(t, carry):
        jA = 2 * t
        jB = 2 * t + 1
        drain_load(jA, 0, lsemA)
        fire_load(jB, 1, lsemB)
        pltpu.sync_copy(vbuf.at[0], accum.at[ridx.at[jA]], add=True)
        drain_load(jB, 1, lsemB)

        @pl.when(t + 1 < NCHUNKS // 2)
        def _():
            fire_load(jA + 2, 0, lsemA)

        pltpu.sync_copy(vbuf.at[1], accum.at[ridx.at[jB]], add=True)
        return carry

    lax.fori_loop(0, NCHUNKS // 2, body, 0)
    plsc.subcore_barrier()
    pltpu.sync_copy(accum.at[pl.ds(sid * ROWS_PER_TILE, ROWS_PER_TILE)], dbuf)
    pltpu.sync_copy(dbuf, part_out.at[cid, pl.ds(sid * ROWS_PER_TILE,
                                                 ROWS_PER_TILE)])


def kernel(node_ft, edge_index, edge_sh, edge_scalars,
           Wq, Wk1, Wk2, Wv1, Wv2, Wdot):
    snd2d = edge_index[0].astype(jnp.int32).reshape(N_EDGES // CHUNK, CHUNK)
    rcv2d = edge_index[1].astype(jnp.int32).reshape(N_EDGES // CHUNK, CHUNK)

    Wu = (Wq @ Wdot) * (1.0 / (jnp.sqrt(32.0) * 16.0))
    Rm = jnp.repeat(jnp.eye(D_IN, dtype=jnp.float32), D_OUT, axis=1)
    Sm = jnp.tile(jnp.eye(D_OUT, dtype=jnp.float32), (D_IN, 1))
    zeros = jnp.zeros((ROWS_PER_TILE, PACK), jnp.float32)

    u = pl.pallas_call(
        _node_proj_body,
        out_shape=jax.ShapeDtypeStruct((N_NODES, 16), jnp.float32),
    )(node_ft, Wu)

    xs_g, qr_g = _sc_gather(node_ft, u, snd2d, rcv2d)

    grid = (N_EDGES // BE,)
    packed = pl.pallas_call(
        _edge_body,
        grid=grid,
        in_specs=[
            pl.BlockSpec((BE, D_IN), lambda i: (i, 0)),
            pl.BlockSpec((BE, D_OUT), lambda i: (i, 0)),
            pl.BlockSpec((BE, 16), lambda i: (i, 0)),
            pl.BlockSpec((BE, 1), lambda i: (i, 0)),
            pl.BlockSpec((16, 32), lambda i: (0, 0)),
            pl.BlockSpec((32, 512), lambda i: (0, 0)),
            pl.BlockSpec((16, 32), lambda i: (0, 0)),
            pl.BlockSpec((32, 512), lambda i: (0, 0)),
            pl.BlockSpec((32, 512), lambda i: (0, 0)),
            pl.BlockSpec((512, 16), lambda i: (0, 0)),
        ],
        out_specs=pl.BlockSpec((BE, PACK), lambda i: (i, 0)),
        out_shape=jax.ShapeDtypeStruct((N_EDGES, PACK), jnp.float32),
    )(xs_g, qr_g, edge_scalars, edge_sh, Wk1, Wk2, Wv1, Wv2, Rm, Sm)

    part = _sc_scatter(packed, rcv2d, zeros)

    out = pl.pallas_call(
        _final_body,
        out_shape=jax.ShapeDtypeStruct((N_NODES, D_OUT), jnp.float32),
    )(part[0], part[1])
    return out


# speedup vs baseline: 4.5766x; 1.1011x over previous
import functools

import jax
import jax.numpy as jnp
from jax import lax
from jax.experimental import pallas as pl
from jax.experimental.pallas import tpu as pltpu
from jax.experimental.pallas import tpu_sc as plsc

N_NODES = 10000
N_EDGES = 160000
D_IN = 32
D_OUT = 16
PACK = 32

NC = 2
NS = 16
NW = NC * NS
CHUNK = 125
BE = 2000
EPW = N_EDGES // NW
NCHUNKS = EPW // CHUNK
KG = 4
NGROUPS = NCHUNKS // KG
ROWS_PER_TILE = N_NODES // NS

_mesh = plsc.VectorSubcoreMesh(core_axis_name="c", subcore_axis_name="s")


def _node_proj_body(nf_ref, wu_ref, u_ref):
    u_ref[...] = jnp.dot(nf_ref[...], wu_ref[...],
                         preferred_element_type=jnp.float32)


def _edge_body(xs_ref, qr_ref, es_ref, sh_ref, wk1_ref, wk2_ref,
               wv1_ref, wv2_ref, s_ref, out_ref):
    xs = (xs_ref[...] * sh_ref[...]).astype(jnp.bfloat16)
    xt = jnp.tile(xs, (1, D_OUT))
    es = es_ref[...].astype(jnp.bfloat16)
    sp = s_ref[...]
    hk = jax.nn.silu(jnp.dot(es, wk1_ref[...],
                             preferred_element_type=jnp.float32) * 0.25)
    wk = jnp.dot(hk.astype(jnp.bfloat16), wk2_ref[...],
                 preferred_element_type=jnp.float32)
    kk = jnp.dot((xt * wk.astype(jnp.bfloat16)), sp,
                 preferred_element_type=jnp.float32) * (1.0 / 32.0)
    hv = jax.nn.silu(jnp.dot(es, wv1_ref[...],
                             preferred_element_type=jnp.float32) * 0.25)
    wv = jnp.dot(hv.astype(jnp.bfloat16), wv2_ref[...],
                 preferred_element_type=jnp.float32)
    vv = jnp.dot((xt * wv.astype(jnp.bfloat16)), sp,
                 preferred_element_type=jnp.float32) * (1.0 / 32.0)
    logit = jnp.sum(qr_ref[...] * kk, axis=1, keepdims=True)
    expv = jnp.exp(logit)
    sv = jnp.exp(0.5 * logit) * vv
    out_ref[...] = jnp.concatenate(
        [expv, sv, jnp.zeros((BE, PACK - 1 - D_OUT), jnp.float32)], axis=1)


def _final_body(p0_ref, p1_ref, out_ref):
    t = p0_ref[...] + p1_ref[...]
    z = t[:, 0:1]
    s = t[:, 1:1 + D_OUT]
    out_ref[...] = jnp.where(z > 0.0,
                             s * jax.lax.rsqrt(jnp.maximum(z, 1e-30)), 0.0)


@functools.partial(
    pl.kernel,
    out_type=[jax.ShapeDtypeStruct((N_EDGES, D_IN), jnp.float32),
              jax.ShapeDtypeStruct((N_EDGES, D_OUT), jnp.float32)],
    mesh=_mesh,
    scratch_types=[pltpu.VMEM((NCHUNKS, CHUNK), jnp.int32),
                   pltpu.VMEM((NCHUNKS, CHUNK), jnp.int32),
                   pltpu.VMEM((2, KG, CHUNK, D_IN), jnp.float32),
                   pltpu.VMEM((2, KG, CHUNK, D_OUT), jnp.float32),
                   pltpu.SemaphoreType.DMA,
                   pltpu.SemaphoreType.DMA,
                   pltpu.SemaphoreType.DMA,
                   pltpu.SemaphoreType.DMA],
    compiler_params=pltpu.CompilerParams(use_tc_tiling_on_sc=False),
)
def _sc_gather(node_hbm, u_hbm, snd_hbm, rcv_hbm, xs_out, qr_out,
               sidx, ridx, xbuf, qbuf, gsemA, gsemB, ssemA, ssemB):
    wid = lax.axis_index("s") * NC + lax.axis_index("c")
    pltpu.sync_copy(snd_hbm.at[pl.ds(wid * NCHUNKS, NCHUNKS)], sidx)
    pltpu.sync_copy(rcv_hbm.at[pl.ds(wid * NCHUNKS, NCHUNKS)], ridx)
    ebase = wid * EPW

    def fire_gathers(g, bank, gsem):
        for b in range(KG):
            j = g * KG + b
            pltpu.async_copy(node_hbm.at[sidx.at[j]], xbuf.at[bank, b], gsem)
            pltpu.async_copy(u_hbm.at[ridx.at[j]], qbuf.at[bank, b], gsem)

    def drain_gathers(g, bank, gsem):
        for b in range(KG):
            j = g * KG + b
            pltpu.make_async_copy(node_hbm.at[sidx.at[j]], xbuf.at[bank, b],
                                  gsem).wait()
            pltpu.make_async_copy(u_hbm.at[ridx.at[j]], qbuf.at[bank, b],
                                  gsem).wait()

    def fire_stores(g, bank, ssem):
        for b in range(KG):
            j = g * KG + b
            dst = xs_out.at[pl.ds(ebase + j * CHUNK, CHUNK)]
            pltpu.async_copy(xbuf.at[bank, b], dst, ssem)
            dq = qr_out.at[pl.ds(ebase + j * CHUNK, CHUNK)]
            pltpu.async_copy(qbuf.at[bank, b], dq, ssem)

    def drain_stores(g, bank, ssem):
        for b in range(KG):
            j = g * KG + b
            dst = xs_out.at[pl.ds(ebase + j * CHUNK, CHUNK)]
            pltpu.make_async_copy(xbuf.at[bank, b], dst, ssem).wait()
            dq = qr_out.at[pl.ds(ebase + j * CHUNK, CHUNK)]
            pltpu.make_async_copy(qbuf.at[bank, b], dq, ssem).wait()

    fire_gathers(0, 0, gsemA)

    def body(t, carry):
        gA = 2 * t
        gB = 2 * t + 1
        drain_gathers(gA, 0, gsemA)
        fire_stores(gA, 0, ssemA)

        @pl.when(t > 0)
        def _():
            drain_stores(gB - 2, 1, ssemB)

        fire_gathers(gB, 1, gsemB)
        drain_stores(gA, 0, ssemA)

        @pl.when(t + 1 < NGROUPS // 2)
        def _():
            fire_gathers(gA + 2, 0, gsemA)

        drain_gathers(gB, 1, gsemB)
        fire_stores(gB, 1, ssemB)
        return carry

    lax.fori_loop(0, NGROUPS // 2, body, 0)
    drain_stores(NGROUPS - 1, 1, ssemB)


@functools.partial(
    pl.kernel,
    out_type=jax.ShapeDtypeStruct((NC, N_NODES, PACK), jnp.float32),
    mesh=_mesh,
    scratch_types=[pltpu.VMEM((NCHUNKS, CHUNK), jnp.int32),
                   pltpu.VMEM((2, CHUNK, PACK), jnp.float32),
                   pltpu.VMEM((ROWS_PER_TILE, PACK), jnp.float32),
                   pltpu.VMEM_SHARED((N_NODES, PACK), jnp.float32),
                   pltpu.SemaphoreType.DMA,
                   pltpu.SemaphoreType.DMA],
    compiler_params=pltpu.CompilerParams(use_tc_tiling_on_sc=False),
)
def _sc_scatter(packed_hbm, rcv_hbm, zeros_hbm, part_out,
                ridx, vbuf, dbuf, accum, lsemA, lsemB):
    cid = lax.axis_index("c")
    sid = lax.axis_index("s")
    wid = sid * NC + cid
    pltpu.sync_copy(zeros_hbm, dbuf)
    pltpu.sync_copy(dbuf, accum.at[pl.ds(sid * ROWS_PER_TILE, ROWS_PER_TILE)])
    plsc.subcore_barrier()
    pltpu.sync_copy(rcv_hbm.at[pl.ds(wid * NCHUNKS, NCHUNKS)], ridx)
    ebase = wid * EPW

    def fire_load(j, bank, sem):
        pltpu.async_copy(packed_hbm.at[pl.ds(ebase + j * CHUNK, CHUNK)],
                         vbuf.at[bank], sem)

    def drain_load(j, bank, sem):
        pltpu.make_async_copy(packed_hbm.at[pl.ds(ebase + j * CHUNK, CHUNK)],
                              vbuf.at[bank], sem).wait()

    fire_load(0, 0, lsemA)

    def body(t, carry):
        jA = 2 * t
        jB = 2 * t + 1
        drain_load(jA, 0, lsemA)
        fire_load(jB, 1, lsemB)
        pltpu.sync_copy(vbuf.at[0], accum.at[ridx.at[jA]], add=True)
        drain_load(jB, 1, lsemB)

        @pl.when(t + 1 < NCHUNKS // 2)
        def _():
            fire_load(jA + 2, 0, lsemA)

        pltpu.sync_copy(vbuf.at[1], accum.at[ridx.at[jB]], add=True)
        return carry

    lax.fori_loop(0, NCHUNKS // 2, body, 0)
    plsc.subcore_barrier()
    pltpu.sync_copy(accum.at[pl.ds(sid * ROWS_PER_TILE, ROWS_PER_TILE)], dbuf)
    pltpu.sync_copy(dbuf, part_out.at[cid, pl.ds(sid * ROWS_PER_TILE,
                                                 ROWS_PER_TILE)])


def kernel(node_ft, edge_index, edge_sh, edge_scalars,
           Wq, Wk1, Wk2, Wv1, Wv2, Wdot):
    snd2d = edge_index[0].astype(jnp.int32).reshape(N_EDGES // CHUNK, CHUNK)
    rcv2d = edge_index[1].astype(jnp.int32).reshape(N_EDGES // CHUNK, CHUNK)

    Wu = (Wq @ Wdot) * (1.0 / (jnp.sqrt(32.0) * 16.0))
    Wk2p = (Wk2.reshape(D_IN, D_IN, D_OUT).transpose(0, 2, 1)
            .reshape(D_IN, D_IN * D_OUT).astype(jnp.bfloat16))
    Wv2p = (Wv2.reshape(D_IN, D_IN, D_OUT).transpose(0, 2, 1)
            .reshape(D_IN, D_IN * D_OUT).astype(jnp.bfloat16))
    Wk1h = Wk1.astype(jnp.bfloat16)
    Wv1h = Wv1.astype(jnp.bfloat16)
    Sp = jnp.repeat(jnp.eye(D_OUT, dtype=jnp.bfloat16), D_IN, axis=0)
    zeros = jnp.zeros((ROWS_PER_TILE, PACK), jnp.float32)

    u = pl.pallas_call(
        _node_proj_body,
        out_shape=jax.ShapeDtypeStruct((N_NODES, 16), jnp.float32),
    )(node_ft, Wu)

    xs_g, qr_g = _sc_gather(node_ft, u, snd2d, rcv2d)

    grid = (N_EDGES // BE,)
    packed = pl.pallas_call(
        _edge_body,
        grid=grid,
        in_specs=[
            pl.BlockSpec((BE, D_IN), lambda i: (i, 0)),
            pl.BlockSpec((BE, D_OUT), lambda i: (i, 0)),
            pl.BlockSpec((BE, 16), lambda i: (i, 0)),
            pl.BlockSpec((BE, 1), lambda i: (i, 0)),
            pl.BlockSpec((16, 32), lambda i: (0, 0)),
            pl.BlockSpec((32, 512), lambda i: (0, 0)),
            pl.BlockSpec((16, 32), lambda i: (0, 0)),
            pl.BlockSpec((32, 512), lambda i: (0, 0)),
            pl.BlockSpec((512, 16), lambda i: (0, 0)),
        ],
        out_specs=pl.BlockSpec((BE, PACK), lambda i: (i, 0)),
        out_shape=jax.ShapeDtypeStruct((N_EDGES, PACK), jnp.float32),
    )(xs_g, qr_g, edge_scalars, edge_sh, Wk1h, Wk2p, Wv1h, Wv2p, Sp)

    part = _sc_scatter(packed, rcv2d, zeros)

    out = pl.pallas_call(
        _final_body,
        out_shape=jax.ShapeDtypeStruct((N_NODES, D_OUT), jnp.float32),
    )(part[0], part[1])
    return out


# speedup vs baseline: 5.4419x; 1.1891x over previous
import functools

import jax
import jax.numpy as jnp
from jax import lax
from jax.experimental import pallas as pl
from jax.experimental.pallas import tpu as pltpu
from jax.experimental.pallas import tpu_sc as plsc

N_NODES = 10000
N_EDGES = 160000
D_IN = 32
D_OUT = 16
PACK = 32

NC = 2
NS = 16
NW = NC * NS
CHUNK = 125
BE = 3200
EPW = N_EDGES // NW
NCHUNKS = EPW // CHUNK
KG = 4
NGROUPS = NCHUNKS // KG
ROWS_PER_TILE = N_NODES // NS

_mesh = plsc.VectorSubcoreMesh(core_axis_name="c", subcore_axis_name="s")


def _node_proj_body(nf_ref, wu_ref, u_ref):
    u_ref[...] = jnp.dot(nf_ref[...], wu_ref[...],
                         preferred_element_type=jnp.float32)


def _edge_body(xs_ref, qr_ref, es_ref, sh_ref, wk1_ref, wk2_ref,
               wv1_ref, wv2_ref, s_ref, out_ref):
    x4 = xs_ref[...]
    xs_flat = jnp.concatenate(
        [x4[:, 32 * q:32 * (q + 1)] for q in range(4)], axis=0)
    q8 = qr_ref[...]
    qr = jnp.concatenate(
        [q8[:, 16 * g:16 * (g + 1)] for g in range(8)], axis=0)
    xs = (xs_flat * sh_ref[...]).astype(jnp.bfloat16)
    xt = jnp.tile(xs, (1, D_OUT))
    es = es_ref[...].astype(jnp.bfloat16)
    sp = s_ref[...]
    hk = jax.nn.silu(jnp.dot(es, wk1_ref[...],
                             preferred_element_type=jnp.float32) * 0.25)
    wk = jnp.dot(hk.astype(jnp.bfloat16), wk2_ref[...],
                 preferred_element_type=jnp.float32)
    kk = jnp.dot((xt * wk.astype(jnp.bfloat16)), sp,
                 preferred_element_type=jnp.float32) * (1.0 / 32.0)
    hv = jax.nn.silu(jnp.dot(es, wv1_ref[...],
                             preferred_element_type=jnp.float32) * 0.25)
    wv = jnp.dot(hv.astype(jnp.bfloat16), wv2_ref[...],
                 preferred_element_type=jnp.float32)
    vv = jnp.dot((xt * wv.astype(jnp.bfloat16)), sp,
                 preferred_element_type=jnp.float32) * (1.0 / 32.0)
    logit = jnp.sum(qr * kk, axis=1, keepdims=True)
    expv = jnp.exp(logit)
    sv = jnp.exp(0.5 * logit) * vv
    packed = jnp.concatenate(
        [expv, sv, jnp.zeros((BE, PACK - 1 - D_OUT), jnp.float32)], axis=1)
    out_ref[...] = jnp.concatenate(
        [packed[800 * q:800 * (q + 1), :] for q in range(4)], axis=1)


def _final_body(p0_ref, p1_ref, out_ref):
    t = p0_ref[...] + p1_ref[...]
    z = t[:, 0:1]
    s = t[:, 1:1 + D_OUT]
    out_ref[...] = jnp.where(z > 0.0,
                             s * jax.lax.rsqrt(jnp.maximum(z, 1e-30)), 0.0)


@functools.partial(
    pl.kernel,
    out_type=[jax.ShapeDtypeStruct((N_EDGES, D_IN), jnp.float32),
              jax.ShapeDtypeStruct((N_EDGES, D_OUT), jnp.float32)],
    mesh=_mesh,
    scratch_types=[pltpu.VMEM((NCHUNKS, CHUNK), jnp.int32),
                   pltpu.VMEM((NCHUNKS, CHUNK), jnp.int32),
                   pltpu.VMEM((2, KG, CHUNK, D_IN), jnp.float32),
                   pltpu.VMEM((2, KG, CHUNK, D_OUT), jnp.float32),
                   pltpu.SemaphoreType.DMA,
                   pltpu.SemaphoreType.DMA,
                   pltpu.SemaphoreType.DMA,
                   pltpu.SemaphoreType.DMA],
    compiler_params=pltpu.CompilerParams(use_tc_tiling_on_sc=False),
)
def _sc_gather(node_hbm, u_hbm, snd_hbm, rcv_hbm, xs_out, qr_out,
               sidx, ridx, xbuf, qbuf, gsemA, gsemB, ssemA, ssemB):
    wid = lax.axis_index("s") * NC + lax.axis_index("c")
    pltpu.sync_copy(snd_hbm.at[pl.ds(wid * NCHUNKS, NCHUNKS)], sidx)
    pltpu.sync_copy(rcv_hbm.at[pl.ds(wid * NCHUNKS, NCHUNKS)], ridx)
    ebase = wid * EPW

    def fire_gathers(g, bank, gsem):
        for b in range(KG):
            j = g * KG + b
            pltpu.async_copy(node_hbm.at[sidx.at[j]], xbuf.at[bank, b], gsem)
            pltpu.async_copy(u_hbm.at[ridx.at[j]], qbuf.at[bank, b], gsem)

    def drain_gathers(g, bank, gsem):
        for b in range(KG):
            j = g * KG + b
            pltpu.make_async_copy(node_hbm.at[sidx.at[j]], xbuf.at[bank, b],
                                  gsem).wait()
            pltpu.make_async_copy(u_hbm.at[ridx.at[j]], qbuf.at[bank, b],
                                  gsem).wait()

    def fire_stores(g, bank, ssem):
        for b in range(KG):
            j = g * KG + b
            dst = xs_out.at[pl.ds(ebase + j * CHUNK, CHUNK)]
            pltpu.async_copy(xbuf.at[bank, b], dst, ssem)
            dq = qr_out.at[pl.ds(ebase + j * CHUNK, CHUNK)]
            pltpu.async_copy(qbuf.at[bank, b], dq, ssem)

    def drain_stores(g, bank, ssem):
        for b in range(KG):
            j = g * KG + b
            dst = xs_out.at[pl.ds(ebase + j * CHUNK, CHUNK)]
            pltpu.make_async_copy(xbuf.at[bank, b], dst, ssem).wait()
            dq = qr_out.at[pl.ds(ebase + j * CHUNK, CHUNK)]
            pltpu.make_async_copy(qbuf.at[bank, b], dq, ssem).wait()

    fire_gathers(0, 0, gsemA)

    def body(t, carry):
        gA = 2 * t
        gB = 2 * t + 1
        drain_gathers(gA, 0, gsemA)
        fire_stores(gA, 0, ssemA)

        @pl.when(t > 0)
        def _():
            drain_stores(gB - 2, 1, ssemB)

        fire_gathers(gB, 1, gsemB)
        drain_stores(gA, 0, ssemA)

        @pl.when(t + 1 < NGROUPS // 2)
        def _():
            fire_gathers(gA + 2, 0, gsemA)

        drain_gathers(gB, 1, gsemB)
        fire_stores(gB, 1, ssemB)
        return carry

    lax.fori_loop(0, NGROUPS // 2, body, 0)
    drain_stores(NGROUPS - 1, 1, ssemB)


@functools.partial(
    pl.kernel,
    out_type=jax.ShapeDtypeStruct((NC, N_NODES, PACK), jnp.float32),
    mesh=_mesh,
    scratch_types=[pltpu.VMEM((NCHUNKS, CHUNK), jnp.int32),
                   pltpu.VMEM((2, CHUNK, PACK), jnp.float32),
                   pltpu.VMEM((ROWS_PER_TILE, PACK), jnp.float32),
                   pltpu.VMEM_SHARED((N_NODES, PACK), jnp.float32),
                   pltpu.SemaphoreType.DMA,
                   pltpu.SemaphoreType.DMA],
    compiler_params=pltpu.CompilerParams(use_tc_tiling_on_sc=False),
)
def _sc_scatter(packed_hbm, rcv_hbm, zeros_hbm, part_out,
                ridx, vbuf, dbuf, accum, lsemA, lsemB):
    cid = lax.axis_index("c")
    sid = lax.axis_index("s")
    wid = sid * NC + cid
    pltpu.sync_copy(zeros_hbm, dbuf)
    pltpu.sync_copy(dbuf, accum.at[pl.ds(sid * ROWS_PER_TILE, ROWS_PER_TILE)])
    plsc.subcore_barrier()
    pltpu.sync_copy(rcv_hbm.at[pl.ds(wid * NCHUNKS, NCHUNKS)], ridx)
    ebase = wid * EPW

    def fire_load(j, bank, sem):
        pltpu.async_copy(packed_hbm.at[pl.ds(ebase + j * CHUNK, CHUNK)],
                         vbuf.at[bank], sem)

    def drain_load(j, bank, sem):
        pltpu.make_async_copy(packed_hbm.at[pl.ds(ebase + j * CHUNK, CHUNK)],
                              vbuf.at[bank], sem).wait()

    fire_load(0, 0, lsemA)

    def body(t, carry):
        jA = 2 * t
        jB = 2 * t + 1
        drain_load(jA, 0, lsemA)
        fire_load(jB, 1, lsemB)
        pltpu.sync_copy(vbuf.at[0], accum.at[ridx.at[jA]], add=True)
        drain_load(jB, 1, lsemB)

        @pl.when(t + 1 < NCHUNKS // 2)
        def _():
            fire_load(jA + 2, 0, lsemA)

        pltpu.sync_copy(vbuf.at[1], accum.at[ridx.at[jB]], add=True)
        return carry

    lax.fori_loop(0, NCHUNKS // 2, body, 0)
    plsc.subcore_barrier()
    pltpu.sync_copy(accum.at[pl.ds(sid * ROWS_PER_TILE, ROWS_PER_TILE)], dbuf)
    pltpu.sync_copy(dbuf, part_out.at[cid, pl.ds(sid * ROWS_PER_TILE,
                                                 ROWS_PER_TILE)])


def kernel(node_ft, edge_index, edge_sh, edge_scalars,
           Wq, Wk1, Wk2, Wv1, Wv2, Wdot):
    snd = edge_index[0].astype(jnp.int32)
    rcv = edge_index[1].astype(jnp.int32)
    nb = N_EDGES // BE
    snd_g = (snd.reshape(nb, 4, BE // 4).transpose(0, 2, 1)
             .reshape(N_EDGES // CHUNK, CHUNK))
    rcv_g = (rcv.reshape(nb, 8, BE // 8).transpose(0, 2, 1)
             .reshape(N_EDGES // CHUNK, CHUNK))
    rcv_s = (rcv.reshape(nb, 4, BE // 4).transpose(0, 2, 1)
             .reshape(N_EDGES // CHUNK, CHUNK))

    Wu = (Wq @ Wdot) * (1.0 / (jnp.sqrt(32.0) * 16.0))
    Wk2p = (Wk2.reshape(D_IN, D_IN, D_OUT).transpose(0, 2, 1)
            .reshape(D_IN, D_IN * D_OUT).astype(jnp.bfloat16))
    Wv2p = (Wv2.reshape(D_IN, D_IN, D_OUT).transpose(0, 2, 1)
            .reshape(D_IN, D_IN * D_OUT).astype(jnp.bfloat16))
    Wk1h = Wk1.astype(jnp.bfloat16)
    Wv1h = Wv1.astype(jnp.bfloat16)
    Sp = jnp.repeat(jnp.eye(D_OUT, dtype=jnp.bfloat16), D_IN, axis=0)
    zeros = jnp.zeros((ROWS_PER_TILE, PACK), jnp.float32)

    u = pl.pallas_call(
        _node_proj_body,
        out_shape=jax.ShapeDtypeStruct((N_NODES, 16), jnp.float32),
    )(node_ft, Wu)

    xs_g, qr_g = _sc_gather(node_ft, u, snd_g, rcv_g)

    xs4 = xs_g.reshape(N_EDGES // 4, 4 * D_IN)
    qr8 = qr_g.reshape(N_EDGES // 8, 8 * D_OUT)
    grid = (N_EDGES // BE,)
    packed4 = pl.pallas_call(
        _edge_body,
        grid=grid,
        in_specs=[
            pl.BlockSpec((BE // 4, 4 * D_IN), lambda i: (i, 0)),
            pl.BlockSpec((BE // 8, 8 * D_OUT), lambda i: (i, 0)),
            pl.BlockSpec((BE, 16), lambda i: (i, 0)),
            pl.BlockSpec((BE, 1), lambda i: (i, 0)),
            pl.BlockSpec((16, 32), lambda i: (0, 0)),
            pl.BlockSpec((32, 512), lambda i: (0, 0)),
            pl.BlockSpec((16, 32), lambda i: (0, 0)),
            pl.BlockSpec((32, 512), lambda i: (0, 0)),
            pl.BlockSpec((512, 16), lambda i: (0, 0)),
        ],
        out_specs=pl.BlockSpec((BE // 4, 4 * PACK), lambda i: (i, 0)),
        out_shape=jax.ShapeDtypeStruct((N_EDGES // 4, 4 * PACK), jnp.float32),
    )(xs4, qr8, edge_scalars, edge_sh, Wk1h, Wk2p, Wv1h, Wv2p, Sp)
    packed = packed4.reshape(N_EDGES, PACK)

    part = _sc_scatter(packed, rcv_s, zeros)

    out = pl.pallas_call(
        _final_body,
        out_shape=jax.ShapeDtypeStruct((N_NODES, D_OUT), jnp.float32),
    )(part[0], part[1])
    return out


# speedup vs baseline: 5.9135x; 1.0867x over previous
import functools

import jax
import jax.numpy as jnp
from jax import lax
from jax.experimental import pallas as pl
from jax.experimental.pallas import tpu as pltpu
from jax.experimental.pallas import tpu_sc as plsc

N_NODES = 10000
N_EDGES = 160000
D_IN = 32
D_OUT = 16
PACK = 32

NC = 2
NS = 16
NW = NC * NS
CHUNK = 125
BE = 3200
EPW = N_EDGES // NW
NCHUNKS = EPW // CHUNK
KG = 4
NGROUPS = NCHUNKS // KG
ROWS_PER_TILE = N_NODES // NS

_mesh = plsc.VectorSubcoreMesh(core_axis_name="c", subcore_axis_name="s")


def _node_proj_body(nf_ref, wu_ref, u_ref):
    u_ref[...] = jnp.dot(nf_ref[...], wu_ref[...],
                         preferred_element_type=jnp.float32)


_DN0 = (((0,), (0,)), ((), ()))


def _edge_body(xs_ref, qr_ref, es_ref, sh_ref, wk1_ref, wk2_ref,
               wv1_ref, wv2_ref, s_ref, one_ref, out_ref):
    x4 = xs_ref[...]
    xs_flat = jnp.concatenate(
        [x4[:, 32 * q:32 * (q + 1)] for q in range(4)], axis=0)
    q8 = qr_ref[...]
    qr = jnp.concatenate(
        [q8[:, 16 * g:16 * (g + 1)] for g in range(8)], axis=0)
    sh_col = jax.lax.dot_general(sh_ref[...], one_ref[...], _DN0,
                                 preferred_element_type=jnp.float32)
    xs = (xs_flat * sh_col).astype(jnp.bfloat16)
    xt = jnp.tile(xs, (1, D_OUT))
    es = es_ref[...].astype(jnp.bfloat16)
    sp = s_ref[...]
    hk = jax.nn.silu(jax.lax.dot_general(
        es, wk1_ref[...], _DN0,
        preferred_element_type=jnp.float32) * 0.25)
    wk = jnp.dot(hk.astype(jnp.bfloat16), wk2_ref[...],
                 preferred_element_type=jnp.float32)
    kk = jnp.dot((xt * wk.astype(jnp.bfloat16)), sp,
                 preferred_element_type=jnp.float32) * (1.0 / 32.0)
    hv = jax.nn.silu(jax.lax.dot_general(
        es, wv1_ref[...], _DN0,
        preferred_element_type=jnp.float32) * 0.25)
    wv = jnp.dot(hv.astype(jnp.bfloat16), wv2_ref[...],
                 preferred_element_type=jnp.float32)
    vv = jnp.dot((xt * wv.astype(jnp.bfloat16)), sp,
                 preferred_element_type=jnp.float32) * (1.0 / 32.0)
    logit = jnp.sum(qr * kk, axis=1, keepdims=True)
    expv = jnp.exp(logit)
    sv = jnp.exp(0.5 * logit) * vv
    packed = jnp.concatenate(
        [expv, sv, jnp.zeros((BE, PACK - 1 - D_OUT), jnp.float32)], axis=1)
    out_ref[...] = jnp.concatenate(
        [packed[800 * q:800 * (q + 1), :] for q in range(4)], axis=1)


def _final_body(p0_ref, p1_ref, out_ref):
    t = p0_ref[...] + p1_ref[...]
    z = t[:, 0:1]
    s = t[:, 1:1 + D_OUT]
    out_ref[...] = jnp.where(z > 0.0,
                             s * jax.lax.rsqrt(jnp.maximum(z, 1e-30)), 0.0)


@functools.partial(
    pl.kernel,
    out_type=[jax.ShapeDtypeStruct((N_EDGES, D_IN), jnp.float32),
              jax.ShapeDtypeStruct((N_EDGES, D_OUT), jnp.float32)],
    mesh=_mesh,
    scratch_types=[pltpu.VMEM((NCHUNKS, CHUNK), jnp.int32),
                   pltpu.VMEM((NCHUNKS, CHUNK), jnp.int32),
                   pltpu.VMEM((2, KG, CHUNK, D_IN), jnp.float32),
                   pltpu.VMEM((2, KG, CHUNK, D_OUT), jnp.float32),
                   pltpu.SemaphoreType.DMA,
                   pltpu.SemaphoreType.DMA,
                   pltpu.SemaphoreType.DMA,
                   pltpu.SemaphoreType.DMA],
    compiler_params=pltpu.CompilerParams(use_tc_tiling_on_sc=False),
)
def _sc_gather(node_hbm, u_hbm, snd_hbm, rcv_hbm, xs_out, qr_out,
               sidx, ridx, xbuf, qbuf, gsemA, gsemB, ssemA, ssemB):
    wid = lax.axis_index("s") * NC + lax.axis_index("c")
    pltpu.sync_copy(snd_hbm.at[pl.ds(wid * NCHUNKS, NCHUNKS)], sidx)
    pltpu.sync_copy(rcv_hbm.at[pl.ds(wid * NCHUNKS, NCHUNKS)], ridx)
    ebase = wid * EPW

    def fire_gathers(g, bank, gsem):
        for b in range(KG):
            j = g * KG + b
            pltpu.async_copy(node_hbm.at[sidx.at[j]], xbuf.at[bank, b], gsem)
            pltpu.async_copy(u_hbm.at[ridx.at[j]], qbuf.at[bank, b], gsem)

    def drain_gathers(g, bank, gsem):
        for b in range(KG):
            j = g * KG + b
            pltpu.make_async_copy(node_hbm.at[sidx.at[j]], xbuf.at[bank, b],
                                  gsem).wait()
            pltpu.make_async_copy(u_hbm.at[ridx.at[j]], qbuf.at[bank, b],
                                  gsem).wait()

    def fire_stores(g, bank, ssem):
        for b in range(KG):
            j = g * KG + b
            dst = xs_out.at[pl.ds(ebase + j * CHUNK, CHUNK)]
            pltpu.async_copy(xbuf.at[bank, b], dst, ssem)
            dq = qr_out.at[pl.ds(ebase + j * CHUNK, CHUNK)]
            pltpu.async_copy(qbuf.at[bank, b], dq, ssem)

    def drain_stores(g, bank, ssem):
        for b in range(KG):
            j = g * KG + b
            dst = xs_out.at[pl.ds(ebase + j * CHUNK, CHUNK)]
            pltpu.make_async_copy(xbuf.at[bank, b], dst, ssem).wait()
            dq = qr_out.at[pl.ds(ebase + j * CHUNK, CHUNK)]
            pltpu.make_async_copy(qbuf.at[bank, b], dq, ssem).wait()

    fire_gathers(0, 0, gsemA)

    def body(t, carry):
        gA = 2 * t
        gB = 2 * t + 1
        drain_gathers(gA, 0, gsemA)
        fire_stores(gA, 0, ssemA)

        @pl.when(t > 0)
        def _():
            drain_stores(gB - 2, 1, ssemB)

        fire_gathers(gB, 1, gsemB)
        drain_stores(gA, 0, ssemA)

        @pl.when(t + 1 < NGROUPS // 2)
        def _():
            fire_gathers(gA + 2, 0, gsemA)

        drain_gathers(gB, 1, gsemB)
        fire_stores(gB, 1, ssemB)
        return carry

    lax.fori_loop(0, NGROUPS // 2, body, 0)
    drain_stores(NGROUPS - 1, 1, ssemB)


@functools.partial(
    pl.kernel,
    out_type=jax.ShapeDtypeStruct((NC, N_NODES, PACK), jnp.float32),
    mesh=_mesh,
    scratch_types=[pltpu.VMEM((NCHUNKS, CHUNK), jnp.int32),
                   pltpu.VMEM((2, CHUNK, PACK), jnp.float32),
                   pltpu.VMEM((ROWS_PER_TILE, PACK), jnp.float32),
                   pltpu.VMEM_SHARED((N_NODES, PACK), jnp.float32),
                   pltpu.SemaphoreType.DMA,
                   pltpu.SemaphoreType.DMA],
    compiler_params=pltpu.CompilerParams(use_tc_tiling_on_sc=False),
)
def _sc_scatter(packed_hbm, rcv_hbm, zeros_hbm, part_out,
                ridx, vbuf, dbuf, accum, lsemA, lsemB):
    cid = lax.axis_index("c")
    sid = lax.axis_index("s")
    wid = sid * NC + cid
    pltpu.sync_copy(zeros_hbm, dbuf)
    pltpu.sync_copy(dbuf, accum.at[pl.ds(sid * ROWS_PER_TILE, ROWS_PER_TILE)])
    plsc.subcore_barrier()
    pltpu.sync_copy(rcv_hbm.at[pl.ds(wid * NCHUNKS, NCHUNKS)], ridx)
    ebase = wid * EPW

    def fire_load(j, bank, sem):
        pltpu.async_copy(packed_hbm.at[pl.ds(ebase + j * CHUNK, CHUNK)],
                         vbuf.at[bank], sem)

    def drain_load(j, bank, sem):
        pltpu.make_async_copy(packed_hbm.at[pl.ds(ebase + j * CHUNK, CHUNK)],
                              vbuf.at[bank], sem).wait()

    fire_load(0, 0, lsemA)

    def body(t, carry):
        jA = 2 * t
        jB = 2 * t + 1
        drain_load(jA, 0, lsemA)
        fire_load(jB, 1, lsemB)
        pltpu.sync_copy(vbuf.at[0], accum.at[ridx.at[jA]], add=True)
        drain_load(jB, 1, lsemB)

        @pl.when(t + 1 < NCHUNKS // 2)
        def _():
            fire_load(jA + 2, 0, lsemA)

        pltpu.sync_copy(vbuf.at[1], accum.at[ridx.at[jB]], add=True)
        return carry

    lax.fori_loop(0, NCHUNKS // 2, body, 0)
    plsc.subcore_barrier()
    pltpu.sync_copy(accum.at[pl.ds(sid * ROWS_PER_TILE, ROWS_PER_TILE)], dbuf)
    pltpu.sync_copy(dbuf, part_out.at[cid, pl.ds(sid * ROWS_PER_TILE,
                                                 ROWS_PER_TILE)])


def kernel(node_ft, edge_index, edge_sh, edge_scalars,
           Wq, Wk1, Wk2, Wv1, Wv2, Wdot):
    snd = edge_index[0].astype(jnp.int32)
    rcv = edge_index[1].astype(jnp.int32)
    nb = N_EDGES // BE
    snd_g = (snd.reshape(nb, 4, BE // 4).transpose(0, 2, 1)
             .reshape(N_EDGES // CHUNK, CHUNK))
    rcv_g = (rcv.reshape(nb, 8, BE // 8).transpose(0, 2, 1)
             .reshape(N_EDGES // CHUNK, CHUNK))
    rcv_s = (rcv.reshape(nb, 4, BE // 4).transpose(0, 2, 1)
             .reshape(N_EDGES // CHUNK, CHUNK))

    Wu = (Wq @ Wdot) * (1.0 / (jnp.sqrt(32.0) * 16.0))
    Wk2p = (Wk2.reshape(D_IN, D_IN, D_OUT).transpose(0, 2, 1)
            .reshape(D_IN, D_IN * D_OUT).astype(jnp.bfloat16))
    Wv2p = (Wv2.reshape(D_IN, D_IN, D_OUT).transpose(0, 2, 1)
            .reshape(D_IN, D_IN * D_OUT).astype(jnp.bfloat16))
    Wk1h = Wk1.astype(jnp.bfloat16)
    Wv1h = Wv1.astype(jnp.bfloat16)
    Sp = jnp.repeat(jnp.eye(D_OUT, dtype=jnp.bfloat16), D_IN, axis=0)
    zeros = jnp.zeros((ROWS_PER_TILE, PACK), jnp.float32)

    u = pl.pallas_call(
        _node_proj_body,
        out_shape=jax.ShapeDtypeStruct((N_NODES, 16), jnp.float32),
    )(node_ft, Wu)

    xs_g, qr_g = _sc_gather(node_ft, u, snd_g, rcv_g)

    xs4 = xs_g.reshape(N_EDGES // 4, 4 * D_IN)
    qr8 = qr_g.reshape(N_EDGES // 8, 8 * D_OUT)
    grid = (N_EDGES // BE,)
    packed4 = pl.pallas_call(
        _edge_body,
        grid=grid,
        in_specs=[
            pl.BlockSpec((BE // 4, 4 * D_IN), lambda i: (i, 0)),
            pl.BlockSpec((BE // 8, 8 * D_OUT), lambda i: (i, 0)),
            pl.BlockSpec((16, BE), lambda i: (0, i)),
            pl.BlockSpec((1, BE), lambda i: (0, i)),
            pl.BlockSpec((16, 32), lambda i: (0, 0)),
            pl.BlockSpec((32, 512), lambda i: (0, 0)),
            pl.BlockSpec((16, 32), lambda i: (0, 0)),
            pl.BlockSpec((32, 512), lambda i: (0, 0)),
            pl.BlockSpec((512, 16), lambda i: (0, 0)),
            pl.BlockSpec((1, 1), lambda i: (0, 0)),
        ],
        out_specs=pl.BlockSpec((BE // 4, 4 * PACK), lambda i: (i, 0)),
        out_shape=jax.ShapeDtypeStruct((N_EDGES // 4, 4 * PACK), jnp.float32),
    )(xs4, qr8, edge_scalars.T, edge_sh.T, Wk1h, Wk2p, Wv1h, Wv2p, Sp,
      jnp.ones((1, 1), jnp.float32))
    packed = packed4.reshape(N_EDGES, PACK)

    part = _sc_scatter(packed, rcv_s, zeros)

    out = pl.pallas_call(
        _final_body,
        out_shape=jax.ShapeDtypeStruct((N_NODES, D_OUT), jnp.float32),
    )(part[0], part[1])
    return out
